# baseline (speedup 1.0000x reference)
import jax
import jax.numpy as jnp
from jax.experimental import pallas as pl


def _net_kernel(x_ref, w1_ref, b1_ref, w2_ref, b2_ref, o_ref):
    in_dim = w1_ref.shape[0]
    out_dim = w2_ref.shape[1]
    wf = jnp.dot(w1_ref[...], w2_ref[...], preferred_element_type=jnp.float32)
    bf = jnp.dot(b1_ref[...], w2_ref[...], preferred_element_type=jnp.float32) + b2_ref[...]
    zz = jnp.zeros((in_dim, out_dim), jnp.float32)
    wbig = jnp.concatenate(
        [jnp.concatenate([wf, zz], axis=1), jnp.concatenate([zz, wf], axis=1)],
        axis=0,
    )
    bbig = jnp.concatenate([bf, bf], axis=1)
    xv = x_ref[...]
    n = xv.shape[0]
    x3 = xv.reshape(n // 16, 16, in_dim)
    xa = x3[:, 0:8, :].reshape(n // 2, in_dim)
    xb = x3[:, 8:16, :].reshape(n // 2, in_dim)
    xab = jnp.concatenate([xa, xb], axis=1)
    o_ref[...] = jnp.dot(xab, wbig, preferred_element_type=jnp.float32) + bbig


def kernel(x, edge_index, W1, b1, W2, b2):
    n, _ = x.shape
    hid = W1.shape[1]
    out_dim = W2.shape[1]
    y = pl.pallas_call(
        _net_kernel,
        out_shape=jax.ShapeDtypeStruct((n // 2, 2 * out_dim), x.dtype),
    )(x, W1, b1.reshape(1, hid), W2, b2.reshape(1, out_dim))
    return (
        y.reshape(n // 16, 8, 2, out_dim)
        .transpose(0, 2, 1, 3)
        .reshape(n, out_dim)
    )


# speedup vs baseline: 1.4769x; 1.4769x over previous
import jax
import jax.numpy as jnp
from jax.experimental import pallas as pl


def _net_kernel(x_ref, w1_ref, b1_ref, w2_ref, b2_ref, o_ref):
    in_dim = w1_ref.shape[0]
    out_dim = w2_ref.shape[1]
    wf = jnp.dot(w1_ref[...], w2_ref[...], preferred_element_type=jnp.float32)
    bf = jnp.dot(b1_ref[...], w2_ref[...], preferred_element_type=jnp.float32) + b2_ref[...]
    zz = jnp.zeros((in_dim, out_dim), jnp.float32)
    wbig = jnp.concatenate(
        [jnp.concatenate([wf, zz], axis=1), jnp.concatenate([zz, wf], axis=1)],
        axis=0,
    )
    bbig = jnp.concatenate([bf, bf], axis=1)
    x3 = x_ref.reshape(x_ref.shape[0] // 2, 2, in_dim)
    xa = x3[:, 0, :]
    xb = x3[:, 1, :]
    xab = jnp.concatenate([xa, xb], axis=1)
    o_ref[...] = jnp.dot(xab, wbig, preferred_element_type=jnp.float32) + bbig


def kernel(x, edge_index, W1, b1, W2, b2):
    n, _ = x.shape
    hid = W1.shape[1]
    out_dim = W2.shape[1]
    y = pl.pallas_call(
        _net_kernel,
        out_shape=jax.ShapeDtypeStruct((n // 2, 2 * out_dim), x.dtype),
    )(x, W1, b1.reshape(1, hid), W2, b2.reshape(1, out_dim))
    return y.reshape(n, out_dim)


# speedup vs baseline: 1.7316x; 1.1724x over previous
import jax
import jax.numpy as jnp
from jax.experimental import pallas as pl


def _net_kernel(x_ref, w1_ref, b1_ref, w2_ref, b2_ref, o_ref):
    wf = jnp.dot(w1_ref[...], w2_ref[...], preferred_element_type=jnp.float32)
    bf = jnp.dot(b1_ref[...], w2_ref[...], preferred_element_type=jnp.float32) + b2_ref[...]
    o_ref[...] = jnp.dot(x_ref[...], wf, preferred_element_type=jnp.float32) + bf


def kernel(x, edge_index, W1, b1, W2, b2):
    n, _ = x.shape
    hid = W1.shape[1]
    out_dim = W2.shape[1]
    return pl.pallas_call(
        _net_kernel,
        out_shape=jax.ShapeDtypeStruct((n, out_dim), x.dtype),
    )(x, W1, b1.reshape(1, hid), W2, b2.reshape(1, out_dim))
